# baseline (speedup 1.0000x reference)
import functools
import math

import jax
import jax.numpy as jnp
import numpy as np
from jax import lax
from jax.experimental import pallas as pl
from jax.experimental.pallas import tpu as pltpu
from jax.experimental.pallas import tpu_sc as plsc

TS_LEN = 4096
SEG = 16
SEQ = TS_LEN // SEG
DM = 64
K = 8192
B = 8
NH = 9
NTOK = B * SEQ

K_SC = 2816
K_TC = K - K_SC


def _build_haar_matrix() -> np.ndarray:
    T = np.zeros((NH, SEQ, SEQ), np.float32)
    T[0, :, :] = 2.0 ** -6
    for ii, lvl in enumerate(range(12, 4, -1), start=1):
        segs_per_win = 1 << (lvl - 4)
        half = segs_per_win // 2
        w = 2.0 ** (-lvl / 2.0)
        ncoef = TS_LEN >> lvl
        for k in range(ncoef):
            m0 = k * segs_per_win
            rows = slice(k * segs_per_win, (k + 1) * segs_per_win)
            T[ii, rows, m0:m0 + half] = w
            T[ii, rows, m0 + half:m0 + segs_per_win] = -w
    return T


def _build_pos_emb() -> np.ndarray:
    position = np.arange(SEQ, dtype=np.float32)[:, None]
    div_term = np.exp(np.arange(0, DM, 2).astype(np.float32)
                      * (-math.log(10000.0) / DM))
    pe = np.zeros((SEQ, DM), np.float32)
    pe[:, 0::2] = np.sin(position * div_term)
    pe[:, 1::2] = np.cos(position * div_term)
    return pe


_HAAR_T = _build_haar_matrix()
_POS = _build_pos_emb()


def _ln(v):
    mu = jnp.mean(v, axis=-1, keepdims=True)
    var = jnp.mean((v - mu) * (v - mu), axis=-1, keepdims=True)
    return (v - mu) / jnp.sqrt(var + 1e-5)


def _main_body(x_ref, cbt_ref, whi_ref, wmid_ref, hemb_ref, tm_ref,
               pos_ref, row_ref, dmin_ref, rest_ref):
    segs = x_ref[...]
    smin = jnp.min(segs, axis=1, keepdims=True)
    smax = jnp.max(segs, axis=1, keepdims=True)
    sn = (segs - smin) / (smax - smin + 1e-8)

    d = jnp.zeros((SEQ, K_TC), jnp.float32)
    for j in range(SEG):
        d = d + jnp.abs(sn[:, j:j + 1] - cbt_ref[j:j + 1, :])

    dmin_ref[0] = jnp.min(d, axis=1, keepdims=True)
    idx = jnp.argmin(d, axis=1)[:, None]
    iota = jax.lax.broadcasted_iota(jnp.int32, (SEQ, K_TC), 1)
    onehot = (iota == idx).astype(jnp.bfloat16)
    row_ref[0] = (
        jax.lax.dot_general(onehot, whi_ref[...], (((1,), (0,)), ((), ())),
                            preferred_element_type=jnp.float32)
        + jax.lax.dot_general(onehot, wmid_ref[...], (((1,), (0,)), ((), ())),
                              preferred_element_type=jnp.float32))

    s_col = jnp.sum(segs, axis=1, keepdims=True)
    hacc = jnp.zeros((SEQ, DM), jnp.float32)
    for i in range(NH):
        hm = jax.lax.dot_general(
            tm_ref[i], s_col, (((1,), (0,)), ((), ())),
            precision=jax.lax.Precision.HIGHEST)
        hacc = hacc + hm * hemb_ref[i:i + 1, :]

    rest_ref[0] = _ln(hacc) + pos_ref[...]


_SC_INFO = plsc.get_sparse_core_info()
_NW = _SC_INFO.num_cores * _SC_INFO.num_subcores
_BPW = NTOK // _NW
_NKC = K_SC // 16


@functools.partial(
    pl.kernel,
    mesh=plsc.VectorSubcoreMesh(core_axis_name="c", subcore_axis_name="s"),
    out_type=[
        jax.ShapeDtypeStruct((NTOK,), jnp.float32),
        jax.ShapeDtypeStruct((NTOK,), jnp.int32),
        jax.ShapeDtypeStruct((NTOK, 2 * DM), jnp.float32),
    ],
    scratch_types=[
        pltpu.VMEM((SEG, K_SC), jnp.float32),
        pltpu.VMEM((_BPW, SEG), jnp.float32),
        pltpu.VMEM((_BPW,), jnp.float32),
        pltpu.VMEM((_BPW,), jnp.int32),
        pltpu.VMEM((_BPW,), jnp.int32),
        pltpu.VMEM((_BPW, 2 * DM), jnp.float32),
        pltpu.SemaphoreType.DMA,
    ],
)
def _sc_scan(cbt_hbm, x_hbm, table_hbm, d_hbm, i_hbm, rows_hbm,
             cb_v, x_v, d_v, i_v, ih_v, rows_v, sem):
    wid = lax.axis_index("s") * _SC_INFO.num_cores + lax.axis_index("c")
    base = wid * _BPW
    pltpu.sync_copy(cbt_hbm, cb_v)
    pltpu.sync_copy(x_hbm.at[pl.ds(base, _BPW)], x_v)

    lane = lax.iota(jnp.int32, 16)
    gdn = lax.GatherDimensionNumbers(
        offset_dims=(), collapsed_slice_dims=(0,), start_index_map=(0,))

    def perm(vec, idx):
        return lax.gather(vec, idx.reshape(16, 1), gdn, (1,),
                          mode=lax.GatherScatterMode.PROMISE_IN_BOUNDS)

    def splat(vec, j):
        return perm(vec, jnp.full((16,), j, jnp.int32))

    def allred(vec, op):
        for s in (1, 2, 4, 8):
            vec = op(vec, perm(vec, lane ^ s))
        return vec

    def grp_body(g, _):
        dvec = jnp.zeros((16,), jnp.float32)
        ivec = jnp.zeros((16,), jnp.int32)

        def seg_body(t, carry):
            dvec, ivec = carry
            row = x_v[g * 16 + t, :]
            mn = allred(row, jnp.minimum)
            mx = allred(row, jnp.maximum)
            row = (row - mn) / (mx - mn + 1e-8)
            sp = [splat(row, j) for j in range(SEG)]
            big = jnp.full((16,), 3.0e38, jnp.float32)
            zi = jnp.zeros((16,), jnp.int32)

            def kc_body(kc, kcarry):
                best, bidx = kcarry
                acc = jnp.zeros((16,), jnp.float32)
                for j in range(SEG):
                    acc = acc + jnp.abs(sp[j] - cb_v[j, pl.ds(kc * 16, 16)])
                kvec = lane + (kc * 16 + K_TC)
                m = acc < best
                return jnp.where(m, acc, best), jnp.where(m, kvec, bidx)

            best, bidx = lax.fori_loop(0, _NKC, kc_body, (big, zi))
            dminv = allred(best, jnp.minimum)
            iminv = allred(jnp.where(best == dminv, bidx, 2 * K),
                           jnp.minimum)
            put = lane == t
            return jnp.where(put, dminv, dvec), jnp.where(put, iminv, ivec)

        dvec, ivec = lax.fori_loop(0, 16, seg_body, (dvec, ivec))
        d_v[pl.ds(g * 16, 16)] = dvec
        i_v[pl.ds(g * 16, 16)] = ivec
        ih_v[pl.ds(g * 16, 16)] = jax.lax.shift_right_logical(ivec, 1)
        return 0

    lax.fori_loop(0, _BPW // 16, grp_body, 0)

    pltpu.async_copy(table_hbm.at[ih_v], rows_v, sem).wait()
    pltpu.sync_copy(d_v, d_hbm.at[pl.ds(base, _BPW)])
    pltpu.sync_copy(i_v, i_hbm.at[pl.ds(base, _BPW)])
    pltpu.sync_copy(rows_v, rows_hbm.at[pl.ds(base, _BPW)])


def _epilogue_body(pair_ref, isc_ref, dsc_ref, row_ref, dtc_ref, rest_ref,
                   out_ref):
    pick_par = (isc_ref[...] & 1) == 1
    sc_row = jnp.where(pick_par, pair_ref[:, DM:], pair_ref[:, :DM])
    pick_sc = dsc_ref[...] < dtc_ref[0]
    seq = jnp.where(pick_sc, sc_row, row_ref[0])
    out_ref[0] = _ln(seq + rest_ref[0])


def kernel(X, codebook, word_emb, haar_emb):
    X2 = X.reshape(NTOK, SEG)
    cbT = codebook.T
    w_hi = word_emb[:K_TC].astype(jnp.bfloat16)
    w_mid = ((word_emb[:K_TC] - w_hi.astype(jnp.float32))
             .astype(jnp.bfloat16))
    tm = jnp.asarray(_HAAR_T)
    pos = jnp.asarray(_POS)

    d_sc, i_sc, pairs = _sc_scan(
        cbT[:, K_TC:], X2, word_emb.reshape(K // 2, 2 * DM))

    row_tc, dmin_tc, rest = pl.pallas_call(
        _main_body,
        grid=(B,),
        in_specs=[
            pl.BlockSpec((SEQ, SEG), lambda b: (b, 0)),
            pl.BlockSpec((SEG, K_TC), lambda b: (0, 0)),
            pl.BlockSpec((K_TC, DM), lambda b: (0, 0)),
            pl.BlockSpec((K_TC, DM), lambda b: (0, 0)),
            pl.BlockSpec((NH, DM), lambda b: (0, 0)),
            pl.BlockSpec((NH, SEQ, SEQ), lambda b: (0, 0, 0)),
            pl.BlockSpec((SEQ, DM), lambda b: (0, 0)),
        ],
        out_specs=[
            pl.BlockSpec((1, SEQ, DM), lambda b: (b, 0, 0)),
            pl.BlockSpec((1, SEQ, 1), lambda b: (b, 0, 0)),
            pl.BlockSpec((1, SEQ, DM), lambda b: (b, 0, 0)),
        ],
        out_shape=[
            jax.ShapeDtypeStruct((B, SEQ, DM), jnp.float32),
            jax.ShapeDtypeStruct((B, SEQ, 1), jnp.float32),
            jax.ShapeDtypeStruct((B, SEQ, DM), jnp.float32),
        ],
    )(X2, cbT[:, :K_TC], w_hi, w_mid, haar_emb, tm, pos)

    out = pl.pallas_call(
        _epilogue_body,
        grid=(B,),
        in_specs=[
            pl.BlockSpec((SEQ, 2 * DM), lambda b: (b, 0)),
            pl.BlockSpec((SEQ, 1), lambda b: (b, 0)),
            pl.BlockSpec((SEQ, 1), lambda b: (b, 0)),
            pl.BlockSpec((1, SEQ, DM), lambda b: (b, 0, 0)),
            pl.BlockSpec((1, SEQ, 1), lambda b: (b, 0, 0)),
            pl.BlockSpec((1, SEQ, DM), lambda b: (b, 0, 0)),
        ],
        out_specs=pl.BlockSpec((1, SEQ, DM), lambda b: (b, 0, 0)),
        out_shape=jax.ShapeDtypeStruct((B, SEQ, DM), jnp.float32),
    )(pairs, i_sc.reshape(NTOK, 1), d_sc.reshape(NTOK, 1),
      row_tc, dmin_tc, rest)

    att_mask = jnp.ones((B, SEQ), jnp.int32)
    return out, att_mask


# speedup vs baseline: 1.0133x; 1.0133x over previous
import functools
import math

import jax
import jax.numpy as jnp
import numpy as np
from jax import lax
from jax.experimental import pallas as pl
from jax.experimental.pallas import tpu as pltpu
from jax.experimental.pallas import tpu_sc as plsc

TS_LEN = 4096
SEG = 16
SEQ = TS_LEN // SEG
DM = 64
K = 8192
B = 8
NH = 9
NTOK = B * SEQ

K_SC = 2816
K_TC = K - K_SC


def _build_haar_matrix() -> np.ndarray:
    T = np.zeros((NH, SEQ, SEQ), np.float32)
    T[0, :, :] = 2.0 ** -6
    for ii, lvl in enumerate(range(12, 4, -1), start=1):
        segs_per_win = 1 << (lvl - 4)
        half = segs_per_win // 2
        w = 2.0 ** (-lvl / 2.0)
        ncoef = TS_LEN >> lvl
        for k in range(ncoef):
            m0 = k * segs_per_win
            rows = slice(k * segs_per_win, (k + 1) * segs_per_win)
            T[ii, rows, m0:m0 + half] = w
            T[ii, rows, m0 + half:m0 + segs_per_win] = -w
    return T


def _build_pos_emb() -> np.ndarray:
    position = np.arange(SEQ, dtype=np.float32)[:, None]
    div_term = np.exp(np.arange(0, DM, 2).astype(np.float32)
                      * (-math.log(10000.0) / DM))
    pe = np.zeros((SEQ, DM), np.float32)
    pe[:, 0::2] = np.sin(position * div_term)
    pe[:, 1::2] = np.cos(position * div_term)
    return pe


_HAAR_T = _build_haar_matrix()
_POS = _build_pos_emb()


def _ln(v):
    mu = jnp.mean(v, axis=-1, keepdims=True)
    var = jnp.mean((v - mu) * (v - mu), axis=-1, keepdims=True)
    return (v - mu) / jnp.sqrt(var + 1e-5)


def _main_body(x_ref, cbt_ref, whi_ref, wmid_ref, hemb_ref, tm_ref,
               pos_ref, row_ref, dmin_ref, rest_ref):
    segs = x_ref[...]
    smin = jnp.min(segs, axis=1, keepdims=True)
    smax = jnp.max(segs, axis=1, keepdims=True)
    sn = (segs - smin) / (smax - smin + 1e-8)

    d = jnp.zeros((SEQ, K_TC), jnp.float32)
    for j in range(SEG):
        d = d + jnp.abs(sn[:, j:j + 1] - cbt_ref[j:j + 1, :])

    dmin_ref[0] = jnp.min(d, axis=1, keepdims=True)
    idx = jnp.argmin(d, axis=1)[:, None]
    iota = jax.lax.broadcasted_iota(jnp.int32, (SEQ, K_TC), 1)
    onehot = (iota == idx).astype(jnp.bfloat16)
    row_ref[0] = (
        jax.lax.dot_general(onehot, whi_ref[...], (((1,), (0,)), ((), ())),
                            preferred_element_type=jnp.float32)
        + jax.lax.dot_general(onehot, wmid_ref[...], (((1,), (0,)), ((), ())),
                              preferred_element_type=jnp.float32))

    s_col = jnp.sum(segs, axis=1, keepdims=True)
    hacc = jnp.zeros((SEQ, DM), jnp.float32)
    for i in range(NH):
        hm = jax.lax.dot_general(
            tm_ref[i], s_col, (((1,), (0,)), ((), ())),
            precision=jax.lax.Precision.HIGHEST)
        hacc = hacc + hm * hemb_ref[i:i + 1, :]

    rest_ref[0] = _ln(hacc) + pos_ref[...]


_SC_INFO = plsc.get_sparse_core_info()
_NW = _SC_INFO.num_cores * _SC_INFO.num_subcores
_BPW = NTOK // _NW
_NKC = K_SC // 16


@functools.partial(
    pl.kernel,
    mesh=plsc.VectorSubcoreMesh(core_axis_name="c", subcore_axis_name="s"),
    out_type=[
        jax.ShapeDtypeStruct((NTOK,), jnp.float32),
        jax.ShapeDtypeStruct((NTOK,), jnp.int32),
        jax.ShapeDtypeStruct((NTOK, 2 * DM), jnp.float32),
    ],
    scratch_types=[
        pltpu.VMEM((SEG, K_SC), jnp.float32),
        pltpu.VMEM((_BPW, SEG), jnp.float32),
        pltpu.VMEM((_BPW,), jnp.float32),
        pltpu.VMEM((_BPW,), jnp.int32),
        pltpu.VMEM((_BPW,), jnp.int32),
        pltpu.VMEM((_BPW, 2 * DM), jnp.float32),
        pltpu.SemaphoreType.DMA,
    ],
)
def _sc_scan(cbt_hbm, x_hbm, table_hbm, d_hbm, i_hbm, rows_hbm,
             cb_v, x_v, d_v, i_v, ih_v, rows_v, sem):
    wid = lax.axis_index("s") * _SC_INFO.num_cores + lax.axis_index("c")
    base = wid * _BPW
    pltpu.sync_copy(cbt_hbm, cb_v)
    pltpu.sync_copy(x_hbm.at[pl.ds(base, _BPW)], x_v)

    lane = lax.iota(jnp.int32, 16)
    gdn = lax.GatherDimensionNumbers(
        offset_dims=(), collapsed_slice_dims=(0,), start_index_map=(0,))

    def perm(vec, idx):
        return lax.gather(vec, idx.reshape(16, 1), gdn, (1,),
                          mode=lax.GatherScatterMode.PROMISE_IN_BOUNDS)

    def splat(vec, j):
        return perm(vec, jnp.full((16,), j, jnp.int32))

    def allred(vec, op):
        for s in (1, 2, 4, 8):
            vec = op(vec, perm(vec, lane ^ s))
        return vec

    def grp_body(g, _):
        dvec = jnp.zeros((16,), jnp.float32)
        ivec = jnp.zeros((16,), jnp.int32)

        def seg_body(t, carry):
            dvec, ivec = carry
            row = x_v[g * 16 + t, :]
            mn = allred(row, jnp.minimum)
            mx = allred(row, jnp.maximum)
            row = (row - mn) / (mx - mn + 1e-8)
            sp = [splat(row, j) for j in range(SEG)]
            big = jnp.full((16,), 3.0e38, jnp.float32)
            zi = jnp.zeros((16,), jnp.int32)

            def kc_body(kc, kcarry):
                best, bidx = kcarry
                acc = jnp.zeros((16,), jnp.float32)
                for j in range(SEG):
                    acc = acc + jnp.abs(sp[j] - cb_v[j, pl.ds(kc * 16, 16)])
                kvec = lane + (kc * 16 + K_TC)
                m = acc < best
                return jnp.where(m, acc, best), jnp.where(m, kvec, bidx)

            best, bidx = lax.fori_loop(0, _NKC, kc_body, (big, zi))
            dminv = allred(best, jnp.minimum)
            iminv = allred(jnp.where(best == dminv, bidx, 2 * K),
                           jnp.minimum)
            put = lane == t
            return jnp.where(put, dminv, dvec), jnp.where(put, iminv, ivec)

        dvec, ivec = lax.fori_loop(0, 16, seg_body, (dvec, ivec))
        d_v[pl.ds(g * 16, 16)] = dvec
        i_v[pl.ds(g * 16, 16)] = ivec
        ih_v[pl.ds(g * 16, 16)] = jax.lax.shift_right_logical(ivec, 1)
        return 0

    lax.fori_loop(0, _BPW // 16, grp_body, 0)

    pltpu.async_copy(table_hbm.at[ih_v], rows_v, sem).wait()
    pltpu.sync_copy(d_v, d_hbm.at[pl.ds(base, _BPW)])
    pltpu.sync_copy(i_v, i_hbm.at[pl.ds(base, _BPW)])
    pltpu.sync_copy(rows_v, rows_hbm.at[pl.ds(base, _BPW)])


def _epilogue_body(pair_ref, isc_ref, dsc_ref, row_ref, dtc_ref, rest_ref,
                   out_ref):
    pick_par = (isc_ref[...] & 1) == 1
    sc_row = jnp.where(pick_par, pair_ref[:, DM:], pair_ref[:, :DM])
    pick_sc = dsc_ref[...] < dtc_ref[...]
    seq = jnp.where(pick_sc, sc_row, row_ref[...])
    out_ref[...] = _ln(seq + rest_ref[...])


def kernel(X, codebook, word_emb, haar_emb):
    X2 = X.reshape(NTOK, SEG)
    cbT = codebook.T
    w_hi = word_emb[:K_TC].astype(jnp.bfloat16)
    w_mid = ((word_emb[:K_TC] - w_hi.astype(jnp.float32))
             .astype(jnp.bfloat16))
    tm = jnp.asarray(_HAAR_T)
    pos = jnp.asarray(_POS)

    d_sc, i_sc, pairs = _sc_scan(
        cbT[:, K_TC:], X2, word_emb.reshape(K // 2, 2 * DM))

    row_tc, dmin_tc, rest = pl.pallas_call(
        _main_body,
        grid=(B,),
        in_specs=[
            pl.BlockSpec((SEQ, SEG), lambda b: (b, 0)),
            pl.BlockSpec((SEG, K_TC), lambda b: (0, 0)),
            pl.BlockSpec((K_TC, DM), lambda b: (0, 0)),
            pl.BlockSpec((K_TC, DM), lambda b: (0, 0)),
            pl.BlockSpec((NH, DM), lambda b: (0, 0)),
            pl.BlockSpec((NH, SEQ, SEQ), lambda b: (0, 0, 0)),
            pl.BlockSpec((SEQ, DM), lambda b: (0, 0)),
        ],
        out_specs=[
            pl.BlockSpec((1, SEQ, DM), lambda b: (b, 0, 0)),
            pl.BlockSpec((1, SEQ, 1), lambda b: (b, 0, 0)),
            pl.BlockSpec((1, SEQ, DM), lambda b: (b, 0, 0)),
        ],
        out_shape=[
            jax.ShapeDtypeStruct((B, SEQ, DM), jnp.float32),
            jax.ShapeDtypeStruct((B, SEQ, 1), jnp.float32),
            jax.ShapeDtypeStruct((B, SEQ, DM), jnp.float32),
        ],
    )(X2, cbT[:, :K_TC], w_hi, w_mid, haar_emb, tm, pos)

    out2 = pl.pallas_call(
        _epilogue_body,
        in_specs=[
            pl.BlockSpec((NTOK, 2 * DM), lambda: (0, 0)),
            pl.BlockSpec((NTOK, 1), lambda: (0, 0)),
            pl.BlockSpec((NTOK, 1), lambda: (0, 0)),
            pl.BlockSpec((NTOK, DM), lambda: (0, 0)),
            pl.BlockSpec((NTOK, 1), lambda: (0, 0)),
            pl.BlockSpec((NTOK, DM), lambda: (0, 0)),
        ],
        out_specs=pl.BlockSpec((NTOK, DM), lambda: (0, 0)),
        out_shape=jax.ShapeDtypeStruct((NTOK, DM), jnp.float32),
    )(pairs, i_sc.reshape(NTOK, 1), d_sc.reshape(NTOK, 1),
      row_tc.reshape(NTOK, DM), dmin_tc.reshape(NTOK, 1),
      rest.reshape(NTOK, DM))

    att_mask = jnp.ones((B, SEQ), jnp.int32)
    return out2.reshape(B, SEQ, DM), att_mask


# speedup vs baseline: 1.0448x; 1.0311x over previous
import functools
import math

import jax
import jax.numpy as jnp
import numpy as np
from jax import lax
from jax.experimental import pallas as pl
from jax.experimental.pallas import tpu as pltpu
from jax.experimental.pallas import tpu_sc as plsc

TS_LEN = 4096
SEG = 16
SEQ = TS_LEN // SEG
DM = 64
K = 8192
B = 8
NH = 9
NTOK = B * SEQ

K_SC = 2944
K_TC = K - K_SC


def _build_haar_matrix() -> np.ndarray:
    T = np.zeros((NH, SEQ, SEQ), np.float32)
    T[0, :, :] = 2.0 ** -6
    for ii, lvl in enumerate(range(12, 4, -1), start=1):
        segs_per_win = 1 << (lvl - 4)
        half = segs_per_win // 2
        w = 2.0 ** (-lvl / 2.0)
        ncoef = TS_LEN >> lvl
        for k in range(ncoef):
            m0 = k * segs_per_win
            rows = slice(k * segs_per_win, (k + 1) * segs_per_win)
            T[ii, rows, m0:m0 + half] = w
            T[ii, rows, m0 + half:m0 + segs_per_win] = -w
    return T


def _build_pos_emb() -> np.ndarray:
    position = np.arange(SEQ, dtype=np.float32)[:, None]
    div_term = np.exp(np.arange(0, DM, 2).astype(np.float32)
                      * (-math.log(10000.0) / DM))
    pe = np.zeros((SEQ, DM), np.float32)
    pe[:, 0::2] = np.sin(position * div_term)
    pe[:, 1::2] = np.cos(position * div_term)
    return pe


_HAAR_T = _build_haar_matrix()
_POS = _build_pos_emb()


def _ln(v):
    mu = jnp.mean(v, axis=-1, keepdims=True)
    var = jnp.mean((v - mu) * (v - mu), axis=-1, keepdims=True)
    return (v - mu) / jnp.sqrt(var + 1e-5)


def _main_body(x_ref, cbt_ref, whi_ref, wmid_ref, hemb_ref, tm_ref,
               pos_ref, row_ref, dmin_ref, rest_ref):
    segs = x_ref[0]
    smin = jnp.min(segs, axis=1, keepdims=True)
    smax = jnp.max(segs, axis=1, keepdims=True)
    sn = (segs - smin) / (smax - smin + 1e-8)

    d = jnp.zeros((SEQ, K_TC), jnp.float32)
    for j in range(SEG):
        d = d + jnp.abs(sn[:, j:j + 1] - cbt_ref[j:j + 1, :])

    dmin_ref[0] = jnp.min(d, axis=1, keepdims=True)
    idx = jnp.argmin(d, axis=1)[:, None]
    iota = jax.lax.broadcasted_iota(jnp.int32, (SEQ, K_TC), 1)
    onehot = (iota == idx).astype(jnp.bfloat16)
    row_ref[0] = (
        jax.lax.dot_general(onehot, whi_ref[...], (((1,), (0,)), ((), ())),
                            preferred_element_type=jnp.float32)
        + jax.lax.dot_general(onehot, wmid_ref[...], (((1,), (0,)), ((), ())),
                              preferred_element_type=jnp.float32))

    s_col = jnp.sum(segs, axis=1, keepdims=True)
    hacc = jnp.zeros((SEQ, DM), jnp.float32)
    for i in range(NH):
        hm = jax.lax.dot_general(
            tm_ref[i], s_col, (((1,), (0,)), ((), ())),
            precision=jax.lax.Precision.HIGHEST)
        hacc = hacc + hm * hemb_ref[i:i + 1, :]

    rest_ref[0] = _ln(hacc) + pos_ref[...]


_SC_INFO = plsc.get_sparse_core_info()
_NW = _SC_INFO.num_cores * _SC_INFO.num_subcores
_BPW = NTOK // _NW
_NKC = K_SC // 16


@functools.partial(
    pl.kernel,
    mesh=plsc.VectorSubcoreMesh(core_axis_name="c", subcore_axis_name="s"),
    out_type=[
        jax.ShapeDtypeStruct((NTOK,), jnp.float32),
        jax.ShapeDtypeStruct((NTOK,), jnp.int32),
        jax.ShapeDtypeStruct((NTOK, 2 * DM), jnp.float32),
    ],
    scratch_types=[
        pltpu.VMEM((SEG, K_SC), jnp.float32),
        pltpu.VMEM((_BPW, SEG), jnp.float32),
        pltpu.VMEM((_BPW,), jnp.float32),
        pltpu.VMEM((_BPW,), jnp.int32),
        pltpu.VMEM((_BPW,), jnp.int32),
        pltpu.VMEM((_BPW, 2 * DM), jnp.float32),
        pltpu.SemaphoreType.DMA,
    ],
)
def _sc_scan(cbt_hbm, x_hbm, table_hbm, d_hbm, i_hbm, rows_hbm,
             cb_v, x_v, d_v, i_v, ih_v, rows_v, sem):
    wid = lax.axis_index("s") * _SC_INFO.num_cores + lax.axis_index("c")
    base = wid * _BPW
    pltpu.sync_copy(cbt_hbm, cb_v)
    pltpu.sync_copy(x_hbm.at[pl.ds(base, _BPW)], x_v)

    lane = lax.iota(jnp.int32, 16)
    gdn = lax.GatherDimensionNumbers(
        offset_dims=(), collapsed_slice_dims=(0,), start_index_map=(0,))

    def perm(vec, idx):
        return lax.gather(vec, idx.reshape(16, 1), gdn, (1,),
                          mode=lax.GatherScatterMode.PROMISE_IN_BOUNDS)

    def splat(vec, j):
        return perm(vec, jnp.full((16,), j, jnp.int32))

    def allred(vec, op):
        for s in (1, 2, 4, 8):
            vec = op(vec, perm(vec, lane ^ s))
        return vec

    def grp_body(g, _):
        dvec = jnp.zeros((16,), jnp.float32)
        ivec = jnp.zeros((16,), jnp.int32)

        def seg_body(t, carry):
            dvec, ivec = carry
            row = x_v[g * 16 + t, :]
            mn = allred(row, jnp.minimum)
            mx = allred(row, jnp.maximum)
            row = (row - mn) / (mx - mn + 1e-8)
            sp = [splat(row, j) for j in range(SEG)]
            big = jnp.full((16,), 3.0e38, jnp.float32)
            zi = jnp.zeros((16,), jnp.int32)

            def kc_body(kc, kcarry):
                best, bidx = kcarry
                acc = jnp.zeros((16,), jnp.float32)
                for j in range(SEG):
                    acc = acc + jnp.abs(sp[j] - cb_v[j, pl.ds(kc * 16, 16)])
                kvec = lane + (kc * 16 + K_TC)
                m = acc < best
                return jnp.where(m, acc, best), jnp.where(m, kvec, bidx)

            best, bidx = lax.fori_loop(0, _NKC, kc_body, (big, zi))
            dminv = allred(best, jnp.minimum)
            iminv = allred(jnp.where(best == dminv, bidx, 2 * K),
                           jnp.minimum)
            put = lane == t
            return jnp.where(put, dminv, dvec), jnp.where(put, iminv, ivec)

        dvec, ivec = lax.fori_loop(0, 16, seg_body, (dvec, ivec))
        d_v[pl.ds(g * 16, 16)] = dvec
        i_v[pl.ds(g * 16, 16)] = ivec
        ih_v[pl.ds(g * 16, 16)] = jax.lax.shift_right_logical(ivec, 1)
        return 0

    lax.fori_loop(0, _BPW // 16, grp_body, 0)

    pltpu.async_copy(table_hbm.at[ih_v], rows_v, sem).wait()
    pltpu.sync_copy(d_v, d_hbm.at[pl.ds(base, _BPW)])
    pltpu.sync_copy(i_v, i_hbm.at[pl.ds(base, _BPW)])
    pltpu.sync_copy(rows_v, rows_hbm.at[pl.ds(base, _BPW)])


def _epilogue_body(pair_ref, par_ref, dsc_ref, row_ref, dtc_ref, rest_ref,
                   out_ref):
    pick_par = par_ref[...] == 1
    sc_row = jnp.where(pick_par, pair_ref[:, DM:], pair_ref[:, :DM])
    pick_sc = dsc_ref[...] < dtc_ref[...]
    seq = jnp.where(pick_sc, sc_row, row_ref[...])
    out_ref[...] = _ln(seq + rest_ref[...])


def kernel(X, codebook, word_emb, haar_emb):
    X3 = X.reshape(B, SEQ, SEG)
    cbT = codebook.T
    w_hi = word_emb[:K_TC].astype(jnp.bfloat16)
    w_mid = ((word_emb[:K_TC] - w_hi.astype(jnp.float32))
             .astype(jnp.bfloat16))
    tm = jnp.asarray(_HAAR_T)
    pos = jnp.asarray(_POS)

    d_sc, i_sc, pairs = _sc_scan(
        cbT[:, K_TC:], X.reshape(NTOK, SEG), word_emb.reshape(K // 2, 2 * DM))

    row_tc, dmin_tc, rest = pl.pallas_call(
        _main_body,
        grid=(B,),
        in_specs=[
            pl.BlockSpec((1, SEQ, SEG), lambda b: (b, 0, 0)),
            pl.BlockSpec((SEG, K_TC), lambda b: (0, 0)),
            pl.BlockSpec((K_TC, DM), lambda b: (0, 0)),
            pl.BlockSpec((K_TC, DM), lambda b: (0, 0)),
            pl.BlockSpec((NH, DM), lambda b: (0, 0)),
            pl.BlockSpec((NH, SEQ, SEQ), lambda b: (0, 0, 0)),
            pl.BlockSpec((SEQ, DM), lambda b: (0, 0)),
        ],
        out_specs=[
            pl.BlockSpec((1, SEQ, DM), lambda b: (b, 0, 0)),
            pl.BlockSpec((1, SEQ, 1), lambda b: (b, 0, 0)),
            pl.BlockSpec((1, SEQ, DM), lambda b: (b, 0, 0)),
        ],
        out_shape=[
            jax.ShapeDtypeStruct((B, SEQ, DM), jnp.float32),
            jax.ShapeDtypeStruct((B, SEQ, 1), jnp.float32),
            jax.ShapeDtypeStruct((B, SEQ, DM), jnp.float32),
        ],
    )(X3, cbT[:, :K_TC], w_hi, w_mid, haar_emb, tm, pos)

    out2 = pl.pallas_call(
        _epilogue_body,
        in_specs=[
            pl.BlockSpec((NTOK, 2 * DM), lambda: (0, 0)),
            pl.BlockSpec((NTOK, 1), lambda: (0, 0)),
            pl.BlockSpec((NTOK, 1), lambda: (0, 0)),
            pl.BlockSpec((NTOK, DM), lambda: (0, 0)),
            pl.BlockSpec((NTOK, 1), lambda: (0, 0)),
            pl.BlockSpec((NTOK, DM), lambda: (0, 0)),
        ],
        out_specs=pl.BlockSpec((NTOK, DM), lambda: (0, 0)),
        out_shape=jax.ShapeDtypeStruct((NTOK, DM), jnp.float32),
    )(pairs, (i_sc & 1).reshape(NTOK, 1), d_sc.reshape(NTOK, 1),
      row_tc.reshape(NTOK, DM), dmin_tc.reshape(NTOK, 1),
      rest.reshape(NTOK, DM))

    att_mask = jnp.ones((B, SEQ), jnp.int32)
    return out2.reshape(B, SEQ, DM), att_mask


# speedup vs baseline: 1.0576x; 1.0122x over previous
import functools
import math

import jax
import jax.numpy as jnp
import numpy as np
from jax import lax
from jax.experimental import pallas as pl
from jax.experimental.pallas import tpu as pltpu
from jax.experimental.pallas import tpu_sc as plsc

TS_LEN = 4096
SEG = 16
SEQ = TS_LEN // SEG
DM = 64
K = 8192
B = 8
NH = 9
NTOK = B * SEQ

K_SC = 3072
K_TC = K - K_SC


def _build_haar_matrix() -> np.ndarray:
    T = np.zeros((NH, SEQ, SEQ), np.float32)
    T[0, :, :] = 2.0 ** -6
    for ii, lvl in enumerate(range(12, 4, -1), start=1):
        segs_per_win = 1 << (lvl - 4)
        half = segs_per_win // 2
        w = 2.0 ** (-lvl / 2.0)
        ncoef = TS_LEN >> lvl
        for k in range(ncoef):
            m0 = k * segs_per_win
            rows = slice(k * segs_per_win, (k + 1) * segs_per_win)
            T[ii, rows, m0:m0 + half] = w
            T[ii, rows, m0 + half:m0 + segs_per_win] = -w
    return T


def _build_pos_emb() -> np.ndarray:
    position = np.arange(SEQ, dtype=np.float32)[:, None]
    div_term = np.exp(np.arange(0, DM, 2).astype(np.float32)
                      * (-math.log(10000.0) / DM))
    pe = np.zeros((SEQ, DM), np.float32)
    pe[:, 0::2] = np.sin(position * div_term)
    pe[:, 1::2] = np.cos(position * div_term)
    return pe


_HAAR_T = _build_haar_matrix()
_POS = _build_pos_emb()


def _ln(v):
    mu = jnp.mean(v, axis=-1, keepdims=True)
    var = jnp.mean((v - mu) * (v - mu), axis=-1, keepdims=True)
    return (v - mu) / jnp.sqrt(var + 1e-5)


def _main_body(x_ref, cbt_ref, whi_ref, wmid_ref, hemb_ref, tm_ref,
               pos_ref, row_ref, dmin_ref, rest_ref):
    segs = x_ref[0]
    smin = jnp.min(segs, axis=1, keepdims=True)
    smax = jnp.max(segs, axis=1, keepdims=True)
    sn = (segs - smin) / (smax - smin + 1e-8)

    d = jnp.zeros((SEQ, K_TC), jnp.float32)
    for j in range(SEG):
        d = d + jnp.abs(sn[:, j:j + 1] - cbt_ref[j:j + 1, :])

    dmin_ref[0] = jnp.min(d, axis=1, keepdims=True)
    idx = jnp.argmin(d, axis=1)[:, None]
    iota = jax.lax.broadcasted_iota(jnp.int32, (SEQ, K_TC), 1)
    onehot = (iota == idx).astype(jnp.bfloat16)
    row_ref[0] = (
        jax.lax.dot_general(onehot, whi_ref[...], (((1,), (0,)), ((), ())),
                            preferred_element_type=jnp.float32)
        + jax.lax.dot_general(onehot, wmid_ref[...], (((1,), (0,)), ((), ())),
                              preferred_element_type=jnp.float32))

    s_col = jnp.sum(segs, axis=1, keepdims=True)
    hacc = jnp.zeros((SEQ, DM), jnp.float32)
    for i in range(NH):
        hm = jax.lax.dot_general(
            tm_ref[i], s_col, (((1,), (0,)), ((), ())),
            precision=jax.lax.Precision.HIGHEST)
        hacc = hacc + hm * hemb_ref[i:i + 1, :]

    rest_ref[0] = _ln(hacc) + pos_ref[...]


_SC_INFO = plsc.get_sparse_core_info()
_NW = _SC_INFO.num_cores * _SC_INFO.num_subcores
_BPW = NTOK // _NW
_NKC = K_SC // 16


@functools.partial(
    pl.kernel,
    mesh=plsc.VectorSubcoreMesh(core_axis_name="c", subcore_axis_name="s"),
    out_type=[
        jax.ShapeDtypeStruct((NTOK,), jnp.float32),
        jax.ShapeDtypeStruct((NTOK,), jnp.int32),
        jax.ShapeDtypeStruct((NTOK, 2 * DM), jnp.float32),
    ],
    scratch_types=[
        pltpu.VMEM((SEG, K_SC), jnp.float32),
        pltpu.VMEM((_BPW, SEG), jnp.float32),
        pltpu.VMEM((_BPW,), jnp.float32),
        pltpu.VMEM((_BPW,), jnp.int32),
        pltpu.VMEM((_BPW,), jnp.int32),
        pltpu.VMEM((_BPW, 2 * DM), jnp.float32),
        pltpu.SemaphoreType.DMA,
    ],
)
def _sc_scan(cbt_hbm, x_hbm, table_hbm, d_hbm, i_hbm, rows_hbm,
             cb_v, x_v, d_v, i_v, ih_v, rows_v, sem):
    wid = lax.axis_index("s") * _SC_INFO.num_cores + lax.axis_index("c")
    base = wid * _BPW
    pltpu.sync_copy(cbt_hbm, cb_v)
    pltpu.sync_copy(x_hbm.at[pl.ds(base, _BPW)], x_v)

    lane = lax.iota(jnp.int32, 16)
    gdn = lax.GatherDimensionNumbers(
        offset_dims=(), collapsed_slice_dims=(0,), start_index_map=(0,))

    def perm(vec, idx):
        return lax.gather(vec, idx.reshape(16, 1), gdn, (1,),
                          mode=lax.GatherScatterMode.PROMISE_IN_BOUNDS)

    def splat(vec, j):
        return perm(vec, jnp.full((16,), j, jnp.int32))

    def allred(vec, op):
        for s in (1, 2, 4, 8):
            vec = op(vec, perm(vec, lane ^ s))
        return vec

    def grp_body(g, _):
        dvec = jnp.zeros((16,), jnp.float32)
        ivec = jnp.zeros((16,), jnp.int32)

        def seg_body(t, carry):
            dvec, ivec = carry
            row = x_v[g * 16 + t, :]
            mn = allred(row, jnp.minimum)
            mx = allred(row, jnp.maximum)
            row = (row - mn) / (mx - mn + 1e-8)
            sp = [splat(row, j) for j in range(SEG)]
            big = jnp.full((16,), 3.0e38, jnp.float32)
            zi = jnp.zeros((16,), jnp.int32)

            def kc_body(kc, kcarry):
                best, bidx = kcarry
                acc = jnp.zeros((16,), jnp.float32)
                for j in range(SEG):
                    acc = acc + jnp.abs(sp[j] - cb_v[j, pl.ds(kc * 16, 16)])
                kvec = lane + (kc * 16 + K_TC)
                m = acc < best
                return jnp.where(m, acc, best), jnp.where(m, kvec, bidx)

            best, bidx = lax.fori_loop(0, _NKC, kc_body, (big, zi))
            dminv = allred(best, jnp.minimum)
            iminv = allred(jnp.where(best == dminv, bidx, 2 * K),
                           jnp.minimum)
            put = lane == t
            return jnp.where(put, dminv, dvec), jnp.where(put, iminv, ivec)

        dvec, ivec = lax.fori_loop(0, 16, seg_body, (dvec, ivec))
        d_v[pl.ds(g * 16, 16)] = dvec
        i_v[pl.ds(g * 16, 16)] = ivec
        ih_v[pl.ds(g * 16, 16)] = jax.lax.shift_right_logical(ivec, 1)
        return 0

    lax.fori_loop(0, _BPW // 16, grp_body, 0)

    pltpu.async_copy(table_hbm.at[ih_v], rows_v, sem).wait()
    pltpu.sync_copy(d_v, d_hbm.at[pl.ds(base, _BPW)])
    pltpu.sync_copy(i_v, i_hbm.at[pl.ds(base, _BPW)])
    pltpu.sync_copy(rows_v, rows_hbm.at[pl.ds(base, _BPW)])


def _epilogue_body(pair_ref, par_ref, dsc_ref, row_ref, dtc_ref, rest_ref,
                   out_ref):
    pick_par = par_ref[...] == 1
    sc_row = jnp.where(pick_par, pair_ref[:, DM:], pair_ref[:, :DM])
    pick_sc = dsc_ref[...] < dtc_ref[...]
    seq = jnp.where(pick_sc, sc_row, row_ref[...])
    out_ref[...] = _ln(seq + rest_ref[...])


def kernel(X, codebook, word_emb, haar_emb):
    X3 = X.reshape(B, SEQ, SEG)
    cbT = codebook.T
    w_hi = word_emb[:K_TC].astype(jnp.bfloat16)
    w_mid = ((word_emb[:K_TC] - w_hi.astype(jnp.float32))
             .astype(jnp.bfloat16))
    tm = jnp.asarray(_HAAR_T)
    pos = jnp.asarray(_POS)

    d_sc, i_sc, pairs = _sc_scan(
        cbT[:, K_TC:], X.reshape(NTOK, SEG), word_emb.reshape(K // 2, 2 * DM))

    row_tc, dmin_tc, rest = pl.pallas_call(
        _main_body,
        grid=(B,),
        in_specs=[
            pl.BlockSpec((1, SEQ, SEG), lambda b: (b, 0, 0)),
            pl.BlockSpec((SEG, K_TC), lambda b: (0, 0)),
            pl.BlockSpec((K_TC, DM), lambda b: (0, 0)),
            pl.BlockSpec((K_TC, DM), lambda b: (0, 0)),
            pl.BlockSpec((NH, DM), lambda b: (0, 0)),
            pl.BlockSpec((NH, SEQ, SEQ), lambda b: (0, 0, 0)),
            pl.BlockSpec((SEQ, DM), lambda b: (0, 0)),
        ],
        out_specs=[
            pl.BlockSpec((1, SEQ, DM), lambda b: (b, 0, 0)),
            pl.BlockSpec((1, SEQ, 1), lambda b: (b, 0, 0)),
            pl.BlockSpec((1, SEQ, DM), lambda b: (b, 0, 0)),
        ],
        out_shape=[
            jax.ShapeDtypeStruct((B, SEQ, DM), jnp.float32),
            jax.ShapeDtypeStruct((B, SEQ, 1), jnp.float32),
            jax.ShapeDtypeStruct((B, SEQ, DM), jnp.float32),
        ],
    )(X3, cbT[:, :K_TC], w_hi, w_mid, haar_emb, tm, pos)

    out2 = pl.pallas_call(
        _epilogue_body,
        in_specs=[
            pl.BlockSpec((NTOK, 2 * DM), lambda: (0, 0)),
            pl.BlockSpec((NTOK, 1), lambda: (0, 0)),
            pl.BlockSpec((NTOK, 1), lambda: (0, 0)),
            pl.BlockSpec((NTOK, DM), lambda: (0, 0)),
            pl.BlockSpec((NTOK, 1), lambda: (0, 0)),
            pl.BlockSpec((NTOK, DM), lambda: (0, 0)),
        ],
        out_specs=pl.BlockSpec((NTOK, DM), lambda: (0, 0)),
        out_shape=jax.ShapeDtypeStruct((NTOK, DM), jnp.float32),
    )(pairs, (i_sc & 1).reshape(NTOK, 1), d_sc.reshape(NTOK, 1),
      row_tc.reshape(NTOK, DM), dmin_tc.reshape(NTOK, 1),
      rest.reshape(NTOK, DM))

    att_mask = jnp.ones((B, SEQ), jnp.int32)
    return out2.reshape(B, SEQ, DM), att_mask
